# baseline (speedup 1.0000x reference)
import functools

import jax
import jax.numpy as jnp
from jax.experimental import pallas as pl

E = 8
TOP_K = 3
L0 = 187
B = 128
NUM_CLASSES = 5


def _conv_block(x, w_ref, b_ref, L):
    w = w_ref[...]
    if w.ndim == 4:
        w = w[0]
    cin, cout = w.shape[1], w.shape[2]
    out = jnp.dot(x[0:L].reshape(L * B, cin), w[0],
                  preferred_element_type=jnp.float32)
    out = out + jnp.dot(x[1:1 + L].reshape(L * B, cin), w[1],
                        preferred_element_type=jnp.float32)
    out = out + jnp.dot(x[2:2 + L].reshape(L * B, cin), w[2],
                        preferred_element_type=jnp.float32)
    b = b_ref[...].reshape(-1)
    return (out + b[None, :]).reshape(L, B, cout)


def _pad_l(core, front, back):
    Cout = core.shape[-1]
    z = jnp.zeros((1, core.shape[1], Cout), dtype=core.dtype)
    parts = [z] * front + [core] + [z] * back
    return jnp.concatenate(parts, axis=0)


def _router_kernel(xT_ref, w1_ref, b1_ref, rw_ref, rb_ref, noise_ref,
                   h_ref, g_ref, cv_ref):
    xT = xT_ref[...]
    w1 = w1_ref[...]
    acc = xT[0:L0, :, None] * w1[0][None, None, :]
    acc = acc + xT[1:1 + L0, :, None] * w1[1][None, None, :]
    acc = acc + xT[2:2 + L0, :, None] * w1[2][None, None, :]
    h_core = jnp.maximum(acc + b1_ref[...][None, :, :], 0.0)
    h_ref[0:1] = jnp.zeros((1, B, 16), jnp.float32)
    h_ref[1:1 + L0] = h_core
    h_ref[1 + L0:2 + L0] = jnp.zeros((1, B, 16), jnp.float32)

    pooled = jnp.sum(h_core, axis=0) / float(L0) + noise_ref[...]
    logits = jnp.dot(pooled, rw_ref[...],
                     preferred_element_type=jnp.float32) + rb_ref[...]
    m = jnp.max(logits, axis=-1, keepdims=True)
    p = jnp.exp(logits - m)
    probs = p / jnp.sum(p, axis=-1, keepdims=True)

    iota = jax.lax.broadcasted_iota(jnp.int32, (B, E), 1)
    remaining = probs
    gates = jnp.zeros((B, E), jnp.float32)
    for _ in range(TOP_K):
        mx = jnp.max(remaining, axis=-1, keepdims=True)
        is_mx = remaining >= mx
        idx = jnp.min(jnp.where(is_mx, iota, E), axis=-1, keepdims=True)
        sel = iota == idx
        gates = gates + jnp.where(sel, probs, 0.0)
        remaining = jnp.where(sel, -1.0, remaining)
    g_ref[...] = gates / jnp.sum(gates, axis=-1, keepdims=True)

    mean_probs = jnp.mean(probs, axis=0)
    mu = jnp.mean(mean_probs)
    var = jnp.sum((mean_probs - mu) ** 2) / float(E - 1)
    cv_ref[...] = (var / (mu + 1e-10) ** 2).reshape(1, 1)


def _expert_kernel(h_ref, g_ref,
                   w11_ref, b11_ref, w12_ref, b12_ref,
                   w21_ref, b21_ref, w22_ref, b22_ref,
                   w31_ref, b31_ref, w32_ref, b32_ref,
                   out_ref):
    e = pl.program_id(0)
    h = h_ref[...]
    a = _pad_l(_conv_block(h, w11_ref, b11_ref, 187), 1, 1)
    a = _conv_block(a, w12_ref, b12_ref, 187)
    a = jnp.maximum(a, 0.0)
    a = jnp.max(a[0:186].reshape(93, 2, B, 32), axis=1)
    a = _pad_l(a, 1, 1)

    a = _pad_l(_conv_block(a, w21_ref, b21_ref, 93), 1, 1)
    a = _conv_block(a, w22_ref, b22_ref, 93)
    a = jnp.maximum(a, 0.0)
    a = jnp.max(a[0:92].reshape(46, 2, B, 128), axis=1)
    a = _pad_l(a, 1, 1)

    a = _pad_l(_conv_block(a, w31_ref, b31_ref, 46), 1, 1)
    a = _conv_block(a, w32_ref, b32_ref, 46)
    a = jnp.maximum(a, 0.0)
    a = jnp.max(a[0:46].reshape(23, 2, B, 512), axis=1)
    a = _pad_l(a, 1, 1)

    eiota = jax.lax.broadcasted_iota(jnp.int32, (B, E), 1)
    g = jnp.sum(jnp.where(eiota == e, g_ref[...], 0.0), axis=1)
    contrib = a * g[None, :, None]

    @pl.when(e == 0)
    def _init():
        out_ref[...] = contrib

    @pl.when(e != 0)
    def _acc():
        out_ref[...] += contrib


def _head_kernel(acc_ref, w2_ref, b2_ref, fc1_ref, fb1_ref, fc2_ref, fb2_ref,
                 out_ref):
    a = acc_ref[...]
    y = _conv_block(a, w2_ref, b2_ref, 23)
    y = jnp.maximum(y, 0.0)
    y = jnp.max(y[0:22].reshape(11, 2, B, 1024), axis=1)
    acc2 = jnp.zeros((B, 256), jnp.float32)
    for l in range(11):
        acc2 = acc2 + jnp.dot(y[l], fc1_ref[l],
                              preferred_element_type=jnp.float32)
    acc2 = jnp.maximum(acc2 + fb1_ref[...], 0.0)
    out = jnp.dot(acc2, fc2_ref[...],
                  preferred_element_type=jnp.float32) + fb2_ref[...]
    out_ref[...] = out


@jax.jit
def kernel(x, params):
    f32 = jnp.float32
    xT = jnp.pad(jnp.transpose(x[:, 0, :], (1, 0)), ((1, 1), (0, 0)))
    noise = jax.random.normal(jax.random.key(1), (B, 16), dtype=f32) * 0.05

    w1 = jnp.transpose(params['conv1_w'][:, 0, :], (1, 0))
    b1 = params['conv1_b'].reshape(1, 16)
    rw = jnp.transpose(params['router_w'], (1, 0))
    rb = params['router_b'].reshape(1, E)

    h, G, cv2 = pl.pallas_call(
        _router_kernel,
        out_shape=[
            jax.ShapeDtypeStruct((189, B, 16), f32),
            jax.ShapeDtypeStruct((B, E), f32),
            jax.ShapeDtypeStruct((1, 1), f32),
        ],
    )(xT, w1, b1, rw, rb, noise)

    ep = params['experts']
    def tw(name):
        return jnp.transpose(ep[name], (0, 3, 2, 1))
    def tb(name):
        return ep[name][:, None, :]
    ws = {
        'w11': tw('b1c1_w'), 'b11': tb('b1c1_b'),
        'w12': tw('b1c2_w'), 'b12': tb('b1c2_b'),
        'w21': tw('b2c1_w'), 'b21': tb('b2c1_b'),
        'w22': tw('b2c2_w'), 'b22': tb('b2c2_b'),
        'w31': tw('b3c1_w'), 'b31': tb('b3c1_b'),
        'w32': tw('b3c2_w'), 'b32': tb('b3c2_b'),
    }

    def wspec(arr):
        blk = (1,) + arr.shape[1:]
        nz = (0,) * (arr.ndim - 1)
        return pl.BlockSpec(blk, lambda e, _nz=nz: (e,) + _nz)

    in_specs = [
        pl.BlockSpec((189, B, 16), lambda e: (0, 0, 0)),
        pl.BlockSpec((B, E), lambda e: (0, 0)),
        wspec(ws['w11']), wspec(ws['b11']),
        wspec(ws['w12']), wspec(ws['b12']),
        wspec(ws['w21']), wspec(ws['b21']),
        wspec(ws['w22']), wspec(ws['b22']),
        wspec(ws['w31']), wspec(ws['b31']),
        wspec(ws['w32']), wspec(ws['b32']),
    ]
    acc = pl.pallas_call(
        _expert_kernel,
        grid=(E,),
        in_specs=in_specs,
        out_specs=pl.BlockSpec((25, B, 512), lambda e: (0, 0, 0)),
        out_shape=jax.ShapeDtypeStruct((25, B, 512), f32),
    )(h, G,
      ws['w11'], ws['b11'], ws['w12'], ws['b12'],
      ws['w21'], ws['b21'], ws['w22'], ws['b22'],
      ws['w31'], ws['b31'], ws['w32'], ws['b32'])

    w2 = jnp.transpose(params['conv2_w'], (2, 1, 0))
    b2 = params['conv2_b'].reshape(1, 1024)
    fc1 = jnp.transpose(params['fc1_w'].reshape(256, 1024, 11), (2, 1, 0))
    fb1 = params['fc1_b'].reshape(1, 256)
    fc2 = jnp.transpose(params['fc2_w'], (1, 0))
    fb2 = params['fc2_b'].reshape(1, NUM_CLASSES)

    y = pl.pallas_call(
        _head_kernel,
        out_shape=jax.ShapeDtypeStruct((B, NUM_CLASSES), f32),
    )(acc, w2, b2, fc1, fb1, fc2, fb2)

    return y, cv2[0, 0]


# speedup vs baseline: 1.5834x; 1.5834x over previous
import functools

import jax
import jax.numpy as jnp
from jax.experimental import pallas as pl

E = 8
TOP_K = 3
L0 = 187
B = 128
NUM_CLASSES = 5


def _conv_block(x, w_ref, b_ref, L):
    w = w_ref[...]
    if w.ndim == 4:
        w = w[0]
    cin, cout = w.shape[1], w.shape[2]
    xb = x.astype(jnp.bfloat16)
    if cin <= 128:
        x3 = jnp.concatenate([xb[0:L], xb[1:1 + L], xb[2:2 + L]], axis=2)
        out = jnp.dot(x3.reshape(L * B, 3 * cin), w.reshape(3 * cin, cout),
                      preferred_element_type=jnp.float32)
    else:
        out = jnp.dot(xb[0:L].reshape(L * B, cin), w[0],
                      preferred_element_type=jnp.float32)
        out = out + jnp.dot(xb[1:1 + L].reshape(L * B, cin), w[1],
                            preferred_element_type=jnp.float32)
        out = out + jnp.dot(xb[2:2 + L].reshape(L * B, cin), w[2],
                            preferred_element_type=jnp.float32)
    b = b_ref[...].reshape(-1).astype(jnp.float32)
    return (out + b[None, :]).reshape(L, B, cout)


def _pad_l(core, front, back):
    Cout = core.shape[-1]
    z = jnp.zeros((1, core.shape[1], Cout), dtype=core.dtype)
    parts = [z] * front + [core] + [z] * back
    return jnp.concatenate(parts, axis=0)


def _router_kernel(probs_ref, g_ref, cv_ref):
    probs = probs_ref[...]

    iota = jax.lax.broadcasted_iota(jnp.int32, (B, E), 1)
    remaining = probs
    gates = jnp.zeros((B, E), jnp.float32)
    for _ in range(TOP_K):
        mx = jnp.max(remaining, axis=-1, keepdims=True)
        is_mx = remaining >= mx
        idx = jnp.min(jnp.where(is_mx, iota, E), axis=-1, keepdims=True)
        sel = iota == idx
        gates = gates + jnp.where(sel, probs, 0.0)
        remaining = jnp.where(sel, -1.0, remaining)
    g_ref[...] = gates / jnp.sum(gates, axis=-1, keepdims=True)

    mean_probs = jnp.mean(probs, axis=0)
    mu = jnp.mean(mean_probs)
    var = jnp.sum((mean_probs - mu) ** 2) / float(E - 1)
    cv_ref[...] = (var / (mu + 1e-10) ** 2).reshape(1, 1)


def _expert_kernel(h_ref, g_ref,
                   w11_ref, b11_ref, w12_ref, b12_ref,
                   w21_ref, b21_ref, w22_ref, b22_ref,
                   w31_ref, b31_ref, w32_ref, b32_ref,
                   out_ref):
    e = pl.program_id(0)
    h = h_ref[...]
    a = _pad_l(_conv_block(h, w11_ref, b11_ref, 187), 1, 1)
    a = _conv_block(a, w12_ref, b12_ref, 187)
    a = jnp.maximum(a, 0.0)
    a = jnp.max(a[0:186].reshape(93, 2, B, 32), axis=1)
    a = _pad_l(a, 1, 1)

    a = _pad_l(_conv_block(a, w21_ref, b21_ref, 93), 1, 1)
    a = _conv_block(a, w22_ref, b22_ref, 93)
    a = jnp.maximum(a, 0.0)
    a = jnp.max(a[0:92].reshape(46, 2, B, 128), axis=1)
    a = _pad_l(a, 1, 1)

    a = _pad_l(_conv_block(a, w31_ref, b31_ref, 46), 1, 1)
    a = _conv_block(a, w32_ref, b32_ref, 46)
    a = jnp.maximum(a, 0.0)
    a = jnp.max(a[0:46].reshape(23, 2, B, 512), axis=1)
    a = _pad_l(a, 1, 1)

    eiota = jax.lax.broadcasted_iota(jnp.int32, (B, E), 1)
    g = jnp.sum(jnp.where(eiota == e, g_ref[...], 0.0), axis=1)
    contrib = a * g[None, :, None]

    @pl.when(e == 0)
    def _init():
        out_ref[...] = contrib

    @pl.when(e != 0)
    def _acc():
        out_ref[...] += contrib


def _head_kernel(acc_ref, w2_ref, b2_ref, fc1_ref, fb1_ref, fc2_ref, fb2_ref,
                 out_ref):
    a = acc_ref[...]
    y = _conv_block(a, w2_ref, b2_ref, 23)
    y = jnp.maximum(y, 0.0)
    y = jnp.max(y[0:22].reshape(11, 2, B, 1024), axis=1)
    acc2 = jnp.zeros((B, 256), jnp.float32)
    for l in range(11):
        acc2 = acc2 + jnp.dot(y[l].astype(jnp.bfloat16), fc1_ref[l],
                              preferred_element_type=jnp.float32)
    acc2 = jnp.maximum(acc2 + fb1_ref[...], 0.0)
    out = jnp.dot(acc2, fc2_ref[...],
                  preferred_element_type=jnp.float32) + fb2_ref[...]
    out_ref[...] = out


@jax.jit
def kernel(x, params):
    f32 = jnp.float32
    hx = jax.lax.conv_general_dilated(
        x, params['conv1_w'], window_strides=(1,), padding=[(1, 1)],
        dimension_numbers=('NCH', 'OIH', 'NCH')) + params['conv1_b'][None, :, None]
    h4 = jax.nn.relu(hx)
    pooled = jnp.mean(h4, axis=-1)
    noise = jax.random.normal(jax.random.key(1), pooled.shape,
                              dtype=pooled.dtype) * 0.05
    pooled = pooled + noise
    logits = pooled @ params['router_w'].T + params['router_b']
    probs = jax.nn.softmax(logits, axis=-1)

    h = jnp.pad(jnp.transpose(h4, (2, 0, 1)), ((1, 1), (0, 0), (0, 0)))

    G, cv2 = pl.pallas_call(
        _router_kernel,
        out_shape=[
            jax.ShapeDtypeStruct((B, E), f32),
            jax.ShapeDtypeStruct((1, 1), f32),
        ],
    )(probs)

    ep = params['experts']
    def tw(name):
        return jnp.transpose(ep[name], (0, 3, 2, 1)).astype(jnp.bfloat16)
    def tb(name):
        return ep[name][:, None, :]
    ws = {
        'w11': tw('b1c1_w'), 'b11': tb('b1c1_b'),
        'w12': tw('b1c2_w'), 'b12': tb('b1c2_b'),
        'w21': tw('b2c1_w'), 'b21': tb('b2c1_b'),
        'w22': tw('b2c2_w'), 'b22': tb('b2c2_b'),
        'w31': tw('b3c1_w'), 'b31': tb('b3c1_b'),
        'w32': tw('b3c2_w'), 'b32': tb('b3c2_b'),
    }

    def wspec(arr):
        blk = (1,) + arr.shape[1:]
        nz = (0,) * (arr.ndim - 1)
        return pl.BlockSpec(blk, lambda e, _nz=nz: (e,) + _nz)

    in_specs = [
        pl.BlockSpec((189, B, 16), lambda e: (0, 0, 0)),
        pl.BlockSpec((B, E), lambda e: (0, 0)),
        wspec(ws['w11']), wspec(ws['b11']),
        wspec(ws['w12']), wspec(ws['b12']),
        wspec(ws['w21']), wspec(ws['b21']),
        wspec(ws['w22']), wspec(ws['b22']),
        wspec(ws['w31']), wspec(ws['b31']),
        wspec(ws['w32']), wspec(ws['b32']),
    ]
    acc = pl.pallas_call(
        _expert_kernel,
        grid=(E,),
        in_specs=in_specs,
        out_specs=pl.BlockSpec((25, B, 512), lambda e: (0, 0, 0)),
        out_shape=jax.ShapeDtypeStruct((25, B, 512), f32),
    )(h, G,
      ws['w11'], ws['b11'], ws['w12'], ws['b12'],
      ws['w21'], ws['b21'], ws['w22'], ws['b22'],
      ws['w31'], ws['b31'], ws['w32'], ws['b32'])

    w2 = jnp.transpose(params['conv2_w'], (2, 1, 0)).astype(jnp.bfloat16)
    b2 = params['conv2_b'].reshape(1, 1024)
    fc1 = jnp.transpose(params['fc1_w'].reshape(256, 1024, 11),
                        (2, 1, 0)).astype(jnp.bfloat16)
    fb1 = params['fc1_b'].reshape(1, 256)
    fc2 = jnp.transpose(params['fc2_w'], (1, 0))
    fb2 = params['fc2_b'].reshape(1, NUM_CLASSES)

    y = pl.pallas_call(
        _head_kernel,
        out_shape=jax.ShapeDtypeStruct((B, NUM_CLASSES), f32),
    )(acc, w2, b2, fc1, fb1, fc2, fb2)

    return y, cv2[0, 0]
